# baseline (speedup 1.0000x reference)
import functools

import jax
import jax.numpy as jnp
from jax import lax
from jax.experimental import pallas as pl
from jax.experimental.pallas import tpu as pltpu
from jax.experimental.pallas import tpu_sc as plsc

_N_BINS = 15
_ROWS = 16384
_COLS = 1000
_NEG = -3.0e38

_R_SC = 6144
_NW = 32
_R_W = _R_SC // _NW
_NGRP = _R_W // 16

_R_TC = _ROWS - _R_SC
_CHUNK = 512
_NCHUNK = _R_TC // _CHUNK
_NBUF = 8

_mesh = plsc.VectorSubcoreMesh(core_axis_name="c", subcore_axis_name="s")


@functools.partial(
    pl.kernel,
    out_type=[
        jax.ShapeDtypeStruct((_R_SC,), jnp.float32),
        jax.ShapeDtypeStruct((_R_SC,), jnp.float32),
    ],
    mesh=_mesh,
    scratch_types=[
        pltpu.VMEM((16, _COLS), jnp.float32),
        pltpu.VMEM((16, _COLS), jnp.float32),
        pltpu.VMEM((_R_W,), jnp.int32),
        pltpu.VMEM((_R_W,), jnp.float32),
        pltpu.VMEM((_R_W,), jnp.float32),
        pltpu.SemaphoreType.DMA,
        pltpu.SemaphoreType.DMA,
        pltpu.SemaphoreType.DMA,
    ],
)
def _sc_conf_acc(
    x_hbm,
    lab_hbm,
    conf_hbm,
    acc_hbm,
    buf0,
    buf1,
    labt,
    conft,
    acct,
    sem0,
    sem1,
    osem,
):
    wid = lax.axis_index("s") * 2 + lax.axis_index("c")
    base = wid * _R_W
    bufs = (buf0, buf1)
    sems = (sem0, sem1)

    def start_group(g, k):
        pltpu.make_async_copy(
            x_hbm.at[pl.ds(base + g * 16, 16), :], bufs[k], sems[k]
        ).start()

    pltpu.make_async_copy(lab_hbm.at[pl.ds(base, _R_W)], labt, osem).start()
    start_group(0, 0)
    start_group(1, 1)
    pltpu.make_async_copy(lab_hbm.at[pl.ds(base, _R_W)], labt, osem).wait()

    lanes = lax.iota(jnp.int32, 16)

    def shuffle(v, k):
        idx = jnp.bitwise_xor(lanes, k)
        return lax.gather(
            v,
            idx[:, None],
            dimension_numbers=lax.GatherDimensionNumbers(
                offset_dims=(),
                collapsed_slice_dims=(0,),
                start_index_map=(0,),
            ),
            slice_sizes=(1,),
            mode=lax.GatherScatterMode.PROMISE_IN_BOUNDS,
        )

    def bmax(v):
        for k in (8, 4, 2, 1):
            v = jnp.maximum(v, shuffle(v, k))
        return v

    def bsum(v):
        for k in (8, 4, 2, 1):
            v = v + shuffle(v, k)
        return v

    def process_group(g, k):
        X = bufs[k]
        pltpu.make_async_copy(
            x_hbm.at[pl.ds(base + g * 16, 16), :], X, sems[k]
        ).wait()

        zero16 = jnp.zeros((16,), jnp.float32)
        goff = pl.multiple_of(g * 16, 16)
        lab16 = labt[pl.ds(goff, 16)]

        xlab16 = zero16
        for i in range(16):
            lab_i = lab16[i]
            aligned = pl.multiple_of((lab_i >> 4) << 4, 16)
            vlab = X[i, pl.ds(aligned, 16)]
            cand = jnp.where(lanes == (lab_i & 15), vlab, _NEG)
            xlab16 = jnp.where(lanes == i, bmax(cand), xlab16)

        def row_body(i, carry):
            m16_c, c16_c = carry
            mv = jnp.full((16,), _NEG, jnp.float32)
            for j in range(62):
                mv = jnp.maximum(mv, X[i, pl.ds(16 * j, 16)])
            tail = X[i, pl.ds(_COLS - 16, 16)]
            mv = jnp.maximum(mv, tail)
            m_b = bmax(mv)
            sv = jnp.zeros((16,), jnp.float32)
            for j in range(62):
                sv = sv + jnp.exp(X[i, pl.ds(16 * j, 16)] - m_b)
            tl = jnp.where(lanes < 8, _NEG, tail)
            sv = sv + jnp.exp(tl - m_b)
            s_b = bsum(sv)
            m16_c = jnp.where(lanes == i, m_b, m16_c)
            c16_c = jnp.where(lanes == i, 1.0 / s_b, c16_c)
            return m16_c, c16_c

        m16, conf16 = lax.fori_loop(0, 16, row_body, (zero16, zero16))

        acc16 = jnp.where(xlab16 == m16, 1.0, 0.0).astype(jnp.float32)
        conft[pl.ds(goff, 16)] = conf16
        acct[pl.ds(goff, 16)] = acc16

        @pl.when(g + 2 < _NGRP)
        def _():
            start_group(g + 2, k)

        return 0

    def pair_body(g2, _):
        process_group(g2 * 2, 0)
        process_group(g2 * 2 + 1, 1)
        return 0

    lax.fori_loop(0, _NGRP // 2, pair_body, 0)

    pltpu.make_async_copy(conft, conf_hbm.at[pl.ds(base, _R_W)], osem).start()
    pltpu.make_async_copy(conft, conf_hbm.at[pl.ds(base, _R_W)], osem).wait()
    pltpu.make_async_copy(acct, acc_hbm.at[pl.ds(base, _R_W)], osem).start()
    pltpu.make_async_copy(acct, acc_hbm.at[pl.ds(base, _R_W)], osem).wait()


def _tc_kernel(x_hbm, lab_ref, bnd_ref, out_ref, buf, sems):
    lo = bnd_ref[0:1, :]
    hi = bnd_ref[1:2, :]

    def start_copy(t, slot):
        pltpu.make_async_copy(
            x_hbm.at[pl.ds(_R_SC + t * _CHUNK, _CHUNK), :],
            buf.at[slot],
            sems.at[slot],
        ).start()

    for k in range(_NBUF):
        start_copy(k, k)

    def body(t, carry):
        cnt, cs, as_ = carry
        slot = jax.lax.rem(t, _NBUF)
        pltpu.make_async_copy(
            x_hbm.at[pl.ds(_R_SC + t * _CHUNK, _CHUNK), :],
            buf.at[slot],
            sems.at[slot],
        ).wait()
        x = buf[slot]
        lab = lab_ref[pl.ds(t * _CHUNK, _CHUNK), :]

        m = jnp.max(x, axis=1, keepdims=True)
        s = jnp.sum(jnp.exp(x - m), axis=1, keepdims=True)
        conf = 1.0 / s

        col = jax.lax.broadcasted_iota(jnp.int32, x.shape, 1)
        idx = jnp.min(jnp.where(x == m, col, _COLS), axis=1, keepdims=True)
        acc = (idx == lab).astype(jnp.float32)

        in_bin = ((conf > lo) & (conf <= hi)).astype(jnp.float32)
        cnt = cnt + jnp.sum(in_bin, axis=0, keepdims=True)
        cs = cs + jnp.sum(in_bin * conf, axis=0, keepdims=True)
        as_ = as_ + jnp.sum(in_bin * acc, axis=0, keepdims=True)

        @pl.when(t + _NBUF < _NCHUNK)
        def _():
            start_copy(t + _NBUF, slot)

        return cnt, cs, as_

    zero = jnp.zeros((1, 16), jnp.float32)
    cnt, cs, as_ = jax.lax.fori_loop(0, _NCHUNK, body, (zero, zero, zero))
    out_ref[0:1, :] = cnt
    out_ref[1:2, :] = cs
    out_ref[2:3, :] = as_


def _combine_kernel(tcb_ref, confs_ref, accs_ref, bnd_ref, out_ref):
    lo = bnd_ref[0:1, :]
    hi = bnd_ref[1:2, :]
    conf = confs_ref[...]
    accv = accs_ref[...]
    in_bin = ((conf > lo) & (conf <= hi)).astype(jnp.float32)
    cnt = tcb_ref[0:1, :] + jnp.sum(in_bin, axis=0, keepdims=True)
    cs = tcb_ref[1:2, :] + jnp.sum(in_bin * conf, axis=0, keepdims=True)
    as_ = tcb_ref[2:3, :] + jnp.sum(in_bin * accv, axis=0, keepdims=True)
    prop = cnt / float(_ROWS)
    denom = jnp.maximum(cnt, 1.0)
    gaps = jnp.where(cnt > 0.0, jnp.abs(cs / denom - as_ / denom) * prop, 0.0)
    out_ref[...] = jnp.sum(gaps).reshape(1, 1)


@jax.jit
def _ece(logits, labels):
    labels = labels.astype(jnp.int32)
    labels2 = labels.reshape(_ROWS, 1)
    bb = jnp.linspace(0.0, 1.0, _N_BINS + 1)
    bounds = jnp.stack(
        [
            jnp.concatenate([bb[:-1], jnp.array([2.0], jnp.float32)]),
            jnp.concatenate([bb[1:], jnp.array([2.0], jnp.float32)]),
        ],
        axis=0,
    )

    conf_sc, acc_sc = _sc_conf_acc(logits, labels)

    tc_bins = pl.pallas_call(
        _tc_kernel,
        in_specs=[
            pl.BlockSpec(memory_space=pl.ANY),
            pl.BlockSpec(memory_space=pltpu.VMEM),
            pl.BlockSpec(memory_space=pltpu.VMEM),
        ],
        out_specs=pl.BlockSpec(memory_space=pltpu.VMEM),
        out_shape=jax.ShapeDtypeStruct((3, 16), jnp.float32),
        scratch_shapes=[
            pltpu.VMEM((_NBUF, _CHUNK, _COLS), jnp.float32),
            pltpu.SemaphoreType.DMA((_NBUF,)),
        ],
    )(logits, labels2[_R_SC:], bounds)

    out = pl.pallas_call(
        _combine_kernel,
        out_shape=jax.ShapeDtypeStruct((1, 1), jnp.float32),
    )(tc_bins, conf_sc.reshape(_R_SC, 1), acc_sc.reshape(_R_SC, 1), bounds)
    return out.reshape(1)


def kernel(logits, labels):
    return _ece(logits, labels)


# speedup vs baseline: 1.0923x; 1.0923x over previous
import jax
import jax.numpy as jnp
from jax.experimental import pallas as pl
from jax.experimental.pallas import tpu as pltpu

_N_BINS = 15
_ROWS = 16384
_COLS = 1000
_CBLK = 8
_G = _COLS // _CBLK
_NEG = -3.0e38


def _ece_kernel(x_ref, lab_ref, bnd_ref, out_ref, m_ref, s_ref, i_ref):
    i = pl.program_id(0)

    @pl.when(i == 0)
    def _init():
        m_ref[...] = jnp.full((1, _ROWS), _NEG, jnp.float32)
        s_ref[...] = jnp.zeros((1, _ROWS), jnp.float32)
        i_ref[...] = jnp.zeros((1, _ROWS), jnp.int32)

    x = x_ref[...]
    bm = jnp.max(x, axis=0, keepdims=True)
    col = jax.lax.broadcasted_iota(jnp.int32, x.shape, 0) + _CBLK * i
    bidx = jnp.min(jnp.where(x == bm, col, _COLS), axis=0, keepdims=True)

    m_old = m_ref[...]
    m_new = jnp.maximum(m_old, bm)
    bsum = jnp.sum(jnp.exp(x - m_new), axis=0, keepdims=True)
    s_ref[...] = s_ref[...] * jnp.exp(m_old - m_new) + bsum
    i_ref[...] = jnp.where(bm > m_old, bidx, i_ref[...])
    m_ref[...] = m_new

    @pl.when(i == _G - 1)
    def _finish():
        conf = 1.0 / s_ref[...]
        acc = (i_ref[...] == lab_ref[...]).astype(jnp.float32)
        lo = bnd_ref[:, 0:1]
        hi = bnd_ref[:, 1:2]
        in_bin = ((conf > lo) & (conf <= hi)).astype(jnp.float32)
        cnt = jnp.sum(in_bin, axis=1)
        cs = jnp.sum(in_bin * conf, axis=1)
        as_ = jnp.sum(in_bin * acc, axis=1)
        prop = cnt / float(_ROWS)
        denom = jnp.maximum(cnt, 1.0)
        gaps = jnp.where(
            cnt > 0.0, jnp.abs(cs / denom - as_ / denom) * prop, 0.0
        )
        out_ref[...] = jnp.sum(gaps).reshape(1, 1)


@jax.jit
def _ece(logits, labels):
    xt = logits.T
    labels2 = labels.astype(jnp.int32).reshape(1, _ROWS)
    bb = jnp.linspace(0.0, 1.0, _N_BINS + 1)
    bounds = jnp.stack([bb[:-1], bb[1:]], axis=1)
    out = pl.pallas_call(
        _ece_kernel,
        grid=(_G,),
        in_specs=[
            pl.BlockSpec((_CBLK, _ROWS), lambda i: (i, 0)),
            pl.BlockSpec((1, _ROWS), lambda i: (0, 0)),
            pl.BlockSpec((_N_BINS, 2), lambda i: (0, 0)),
        ],
        out_specs=pl.BlockSpec((1, 1), lambda i: (0, 0)),
        out_shape=jax.ShapeDtypeStruct((1, 1), jnp.float32),
        scratch_shapes=[
            pltpu.VMEM((1, _ROWS), jnp.float32),
            pltpu.VMEM((1, _ROWS), jnp.float32),
            pltpu.VMEM((1, _ROWS), jnp.int32),
        ],
    )(xt, labels2, bounds)
    return out.reshape(1)


def kernel(logits, labels):
    return _ece(logits, labels)


# speedup vs baseline: 3.5999x; 3.2957x over previous
import jax
import jax.numpy as jnp
from jax.experimental import pallas as pl
from jax.experimental.pallas import tpu as pltpu

_N_BINS = 15
_ROWS = 16384
_COLS = 1000
_CBLK = 200
_G = _COLS // _CBLK
_NEG = -3.0e38


def _ece_kernel(x_ref, lab_ref, bnd_ref, out_ref, m_ref, s_ref, i_ref):
    i = pl.program_id(0)

    @pl.when(i == 0)
    def _init():
        m_ref[...] = jnp.full((1, _ROWS), _NEG, jnp.float32)
        s_ref[...] = jnp.zeros((1, _ROWS), jnp.float32)
        i_ref[...] = jnp.zeros((1, _ROWS), jnp.int32)

    x = x_ref[...]
    bm = jnp.max(x, axis=0, keepdims=True)
    col = jax.lax.broadcasted_iota(jnp.int32, x.shape, 0) + _CBLK * i
    bidx = jnp.min(jnp.where(x == bm, col, _COLS), axis=0, keepdims=True)

    m_old = m_ref[...]
    m_new = jnp.maximum(m_old, bm)
    bsum = jnp.sum(jnp.exp(x - m_new), axis=0, keepdims=True)
    s_ref[...] = s_ref[...] * jnp.exp(m_old - m_new) + bsum
    i_ref[...] = jnp.where(bm > m_old, bidx, i_ref[...])
    m_ref[...] = m_new

    @pl.when(i == _G - 1)
    def _finish():
        conf = 1.0 / s_ref[...]
        acc = (i_ref[...] == lab_ref[...]).astype(jnp.float32)
        lo = bnd_ref[:, 0:1]
        hi = bnd_ref[:, 1:2]
        in_bin = ((conf > lo) & (conf <= hi)).astype(jnp.float32)
        cnt = jnp.sum(in_bin, axis=1)
        cs = jnp.sum(in_bin * conf, axis=1)
        as_ = jnp.sum(in_bin * acc, axis=1)
        prop = cnt / float(_ROWS)
        denom = jnp.maximum(cnt, 1.0)
        gaps = jnp.where(
            cnt > 0.0, jnp.abs(cs / denom - as_ / denom) * prop, 0.0
        )
        out_ref[...] = jnp.sum(gaps).reshape(1, 1)


@jax.jit
def _ece(logits, labels):
    xt = logits.T
    labels2 = labels.astype(jnp.int32).reshape(1, _ROWS)
    bb = jnp.linspace(0.0, 1.0, _N_BINS + 1)
    bounds = jnp.stack([bb[:-1], bb[1:]], axis=1)
    out = pl.pallas_call(
        _ece_kernel,
        grid=(_G,),
        in_specs=[
            pl.BlockSpec((_CBLK, _ROWS), lambda i: (i, 0)),
            pl.BlockSpec((1, _ROWS), lambda i: (0, 0)),
            pl.BlockSpec((_N_BINS, 2), lambda i: (0, 0)),
        ],
        out_specs=pl.BlockSpec((1, 1), lambda i: (0, 0)),
        out_shape=jax.ShapeDtypeStruct((1, 1), jnp.float32),
        scratch_shapes=[
            pltpu.VMEM((1, _ROWS), jnp.float32),
            pltpu.VMEM((1, _ROWS), jnp.float32),
            pltpu.VMEM((1, _ROWS), jnp.int32),
        ],
    )(xt, labels2, bounds)
    return out.reshape(1)


def kernel(logits, labels):
    return _ece(logits, labels)


# speedup vs baseline: 4.0650x; 1.1292x over previous
import jax
import jax.numpy as jnp
from jax.experimental import pallas as pl
from jax.experimental.pallas import tpu as pltpu

_N_BINS = 15
_ROWS = 16384
_COLS = 1000
_CBLK = 200
_G = _COLS // _CBLK
_NEG = -3.0e38


def _ece_kernel(x_ref, lab_ref, bnd_ref, out_ref, m_ref, s_ref, i_ref):
    i = pl.program_id(0)

    @pl.when(i == 0)
    def _init():
        m_ref[...] = jnp.full((1, _ROWS), _NEG, jnp.float32)
        s_ref[...] = jnp.zeros((1, _ROWS), jnp.float32)
        i_ref[...] = jnp.zeros((1, _ROWS), jnp.int32)

    x = x_ref[...]
    bm = jnp.max(x, axis=0, keepdims=True)
    col = jax.lax.broadcasted_iota(jnp.int32, x.shape, 0) + _CBLK * i
    bidx = jnp.min(jnp.where(x == bm, col, _COLS), axis=0, keepdims=True)

    m_old = m_ref[...]
    m_new = jnp.maximum(m_old, bm)
    ex = jnp.exp(x - m_new)
    ones = jnp.full((1, _CBLK), 1.0, jnp.float32)
    bsum = jax.lax.dot_general(
        ones,
        ex,
        (((1,), (0,)), ((), ())),
        preferred_element_type=jnp.float32,
    )
    s_ref[...] = s_ref[...] * jnp.exp(m_old - m_new) + bsum
    i_ref[...] = jnp.where(bm > m_old, bidx, i_ref[...])
    m_ref[...] = m_new

    @pl.when(i == _G - 1)
    def _finish():
        conf = 1.0 / s_ref[...]
        acc = (i_ref[...] == lab_ref[...]).astype(jnp.float32)
        lo = bnd_ref[:, 0:1]
        hi = bnd_ref[:, 1:2]
        in_bin = ((conf > lo) & (conf <= hi)).astype(jnp.float32)
        cnt = jnp.sum(in_bin, axis=1)
        cs = jnp.sum(in_bin * conf, axis=1)
        as_ = jnp.sum(in_bin * acc, axis=1)
        prop = cnt / float(_ROWS)
        denom = jnp.maximum(cnt, 1.0)
        gaps = jnp.where(
            cnt > 0.0, jnp.abs(cs / denom - as_ / denom) * prop, 0.0
        )
        out_ref[...] = jnp.sum(gaps).reshape(1, 1)


@jax.jit
def _ece(logits, labels):
    xt = logits.T
    labels2 = labels.astype(jnp.int32).reshape(1, _ROWS)
    bb = jnp.linspace(0.0, 1.0, _N_BINS + 1)
    bounds = jnp.stack([bb[:-1], bb[1:]], axis=1)
    out = pl.pallas_call(
        _ece_kernel,
        grid=(_G,),
        in_specs=[
            pl.BlockSpec((_CBLK, _ROWS), lambda i: (i, 0)),
            pl.BlockSpec((1, _ROWS), lambda i: (0, 0)),
            pl.BlockSpec((_N_BINS, 2), lambda i: (0, 0)),
        ],
        out_specs=pl.BlockSpec((1, 1), lambda i: (0, 0)),
        out_shape=jax.ShapeDtypeStruct((1, 1), jnp.float32),
        scratch_shapes=[
            pltpu.VMEM((1, _ROWS), jnp.float32),
            pltpu.VMEM((1, _ROWS), jnp.float32),
            pltpu.VMEM((1, _ROWS), jnp.int32),
        ],
    )(xt, labels2, bounds)
    return out.reshape(1)


def kernel(logits, labels):
    return _ece(logits, labels)


# speedup vs baseline: 4.0778x; 1.0031x over previous
import jax
import jax.numpy as jnp
from jax.experimental import pallas as pl
from jax.experimental.pallas import tpu as pltpu

_N_BINS = 15
_ROWS = 16384
_COLS = 1000
_CBLK = 200
_G = _COLS // _CBLK
_NEG = -3.0e38


def _ece_kernel(x_ref, lab_ref, bnd_ref, out_ref, m_ref, s_ref, i_ref):
    i = pl.program_id(0)

    @pl.when(i == 0)
    def _init():
        m_ref[...] = jnp.full((1, _ROWS), _NEG, jnp.float32)
        s_ref[...] = jnp.zeros((1, _ROWS), jnp.float32)
        i_ref[...] = jnp.zeros((1, _ROWS), jnp.int32)

    x = x_ref[...]
    bm = jnp.max(x, axis=0, keepdims=True)
    col = jax.lax.broadcasted_iota(jnp.int32, x.shape, 0) + _CBLK * i
    bidx = jnp.min(jnp.where(x == bm, col, _COLS), axis=0, keepdims=True)

    m_old = m_ref[...]
    m_new = jnp.maximum(m_old, bm)
    ex = jnp.exp(x - m_new)
    ones = jnp.full((1, _CBLK), 1.0, jnp.float32)
    bsum = jax.lax.dot_general(
        ones,
        ex,
        (((1,), (0,)), ((), ())),
        preferred_element_type=jnp.float32,
    )
    s_ref[...] = s_ref[...] * jnp.exp(m_old - m_new) + bsum
    i_ref[...] = jnp.where(bm > m_old, bidx, i_ref[...])
    m_ref[...] = m_new

    @pl.when(i == _G - 1)
    def _finish():
        conf = 1.0 / s_ref[...]
        acc = (i_ref[...] == lab_ref[...]).astype(jnp.float32)
        lo = bnd_ref[:, 0:1]
        hi = bnd_ref[:, 1:2]
        in_bin = ((conf > lo) & (conf <= hi)).astype(jnp.float32)

        def lane_sum(v):
            return jax.lax.dot_general(
                in_bin,
                v,
                (((1,), (1,)), ((), ())),
                preferred_element_type=jnp.float32,
            ).reshape(_N_BINS)

        cnt = lane_sum(jnp.full((1, _ROWS), 1.0, jnp.float32))
        cs = lane_sum(conf)
        as_ = lane_sum(acc)
        prop = cnt / float(_ROWS)
        denom = jnp.maximum(cnt, 1.0)
        gaps = jnp.where(
            cnt > 0.0, jnp.abs(cs / denom - as_ / denom) * prop, 0.0
        )
        out_ref[...] = jnp.sum(gaps).reshape(1, 1)


@jax.jit
def _ece(logits, labels):
    xt = logits.T
    labels2 = labels.astype(jnp.int32).reshape(1, _ROWS)
    bb = jnp.linspace(0.0, 1.0, _N_BINS + 1)
    bounds = jnp.stack([bb[:-1], bb[1:]], axis=1)
    out = pl.pallas_call(
        _ece_kernel,
        grid=(_G,),
        in_specs=[
            pl.BlockSpec((_CBLK, _ROWS), lambda i: (i, 0)),
            pl.BlockSpec((1, _ROWS), lambda i: (0, 0)),
            pl.BlockSpec((_N_BINS, 2), lambda i: (0, 0)),
        ],
        out_specs=pl.BlockSpec((1, 1), lambda i: (0, 0)),
        out_shape=jax.ShapeDtypeStruct((1, 1), jnp.float32),
        scratch_shapes=[
            pltpu.VMEM((1, _ROWS), jnp.float32),
            pltpu.VMEM((1, _ROWS), jnp.float32),
            pltpu.VMEM((1, _ROWS), jnp.int32),
        ],
    )(xt, labels2, bounds)
    return out.reshape(1)


def kernel(logits, labels):
    return _ece(logits, labels)
